# baseline (speedup 1.0000x reference)
import functools

import jax
import jax.numpy as jnp
from jax import lax
from jax.experimental import pallas as pl
from jax.experimental.pallas import tpu as pltpu
from jax.experimental.pallas import tpu_sc as plsc

B = 4096
F = 26
V = 100001
E = 32
D = 13
H1 = 128
H2 = 128

NC = 2
NS = 16
NW = NC * NS
B_PER_W = B // NW
CHUNK = 16


ROWS = B * F
R_PER_W = ROWS // NW


def _sc_gather_body(idx_hbm, fm_hbm, lin_hbm, g_out, lv_out,
                    idx_v, rows_v, lin_v, sem_fm, sem_lin):
    wid = lax.axis_index("s") * NC + lax.axis_index("c")
    base = wid * R_PER_W
    pltpu.sync_copy(idx_hbm.at[pl.ds(base, R_PER_W)], idx_v)
    cp_fm = pltpu.async_copy(fm_hbm.at[idx_v], rows_v, sem_fm)
    cp_lin = pltpu.async_copy(lin_hbm.at[idx_v], lin_v, sem_lin)
    cp_fm.wait()
    pltpu.sync_copy(rows_v, g_out.at[pl.ds(base, R_PER_W)])
    cp_lin.wait()
    pltpu.sync_copy(lin_v, lv_out.at[pl.ds(base, R_PER_W)])


@jax.jit
def _sc_gather(flat_idx, fm_flat, lin_flat):
    mesh = plsc.VectorSubcoreMesh(
        core_axis_name="c", subcore_axis_name="s",
        num_cores=NC, num_subcores=NS)
    return pl.kernel(
        _sc_gather_body,
        out_type=(
            jax.ShapeDtypeStruct((ROWS, E), jnp.float32),
            jax.ShapeDtypeStruct((ROWS,), jnp.float32),
        ),
        mesh=mesh,
        scratch_types=[
            pltpu.VMEM((R_PER_W,), jnp.int32),
            pltpu.VMEM((R_PER_W, E), jnp.float32),
            pltpu.VMEM((R_PER_W,), jnp.float32),
            pltpu.SemaphoreType.DMA,
            pltpu.SemaphoreType.DMA,
        ],
        compiler_params=pltpu.CompilerParams(use_tc_tiling_on_sc=False),
    )(flat_idx, fm_flat, lin_flat)


def _tc_body(g_ref, lv_ref, d_ref, w1a_ref, w1b_ref, b1_ref,
             w2_ref, b2_ref, w3_ref, b3_ref, s_ref, out_ref):
    g = g_ref[...]
    sum_v = jnp.dot(g, s_ref[...], preferred_element_type=jnp.float32)
    sq_of_sum = jnp.sum(sum_v * sum_v, axis=1, keepdims=True)
    sum_of_sq = jnp.sum(g * g, axis=1, keepdims=True)
    fm = 0.5 * (sq_of_sum - sum_of_sq)
    lin = jnp.sum(lv_ref[...], axis=1, keepdims=True)
    h = jnp.dot(g, w1a_ref[...], preferred_element_type=jnp.float32)
    h = h + jnp.dot(d_ref[...], w1b_ref[...], preferred_element_type=jnp.float32)
    h = jnp.maximum(h + b1_ref[...], 0.0)
    h = jnp.dot(h, w2_ref[...], preferred_element_type=jnp.float32)
    h = jnp.maximum(h + b2_ref[...], 0.0)
    dp = jnp.sum(h * w3_ref[...], axis=1, keepdims=True)
    out_ref[...] = jax.nn.sigmoid(dp + b3_ref[0, 0] + fm + lin)


def _tc_deepfm(g, lv, dense, w1a, w1b, b1, w2, b2, w3r, b3, smat, bs):
    grid = (B // bs,)
    full = lambda shape: pl.BlockSpec(shape, lambda i: (0, 0))
    return pl.pallas_call(
        _tc_body,
        grid=grid,
        in_specs=[
            pl.BlockSpec((bs, F * E), lambda i: (i, 0)),
            pl.BlockSpec((bs, F), lambda i: (i, 0)),
            pl.BlockSpec((bs, D), lambda i: (i, 0)),
            full((F * E, H1)),
            full((D, H1)),
            full((1, H1)),
            full((H1, H2)),
            full((1, H2)),
            full((1, H2)),
            full((1, 1)),
            full((F * E, E)),
        ],
        out_specs=pl.BlockSpec((bs, 1), lambda i: (i, 0)),
        out_shape=jax.ShapeDtypeStruct((B, 1), jnp.float32),
    )(g, lv, dense, w1a, w1b, b1, w2, b2, w3r, b3, smat)


def kernel(sparse_indices, dense, fm_tables, lin_tables, W1, b1, W2, b2, W3, b3):
    idx = sparse_indices.astype(jnp.int32)
    flat_idx = (idx + (jnp.arange(F, dtype=jnp.int32) * V)[None, :]).reshape(ROWS)
    fm_flat = fm_tables.reshape(F * V, E)
    lin_flat = lin_tables.reshape(F * V)
    g_rows, lin_rows = _sc_gather(flat_idx, fm_flat, lin_flat)
    g = g_rows.reshape(B, F * E)
    lv = lin_rows.reshape(B, F)
    w1a = W1[:F * E]
    w1b = W1[F * E:]
    smat = jnp.tile(jnp.eye(E, dtype=jnp.float32), (F, 1))
    return _tc_deepfm(
        g, lv, dense,
        w1a, w1b, b1.reshape(1, H1),
        W2, b2.reshape(1, H2),
        W3.reshape(1, H2), b3.reshape(1, 1),
        smat, bs=512)


# speedup vs baseline: 3.9855x; 3.9855x over previous
import functools

import jax
import jax.numpy as jnp
from jax import lax
from jax.experimental import pallas as pl
from jax.experimental.pallas import tpu as pltpu
from jax.experimental.pallas import tpu_sc as plsc

B = 4096
F = 26
V = 100001
E = 32
D = 13
H1 = 128
H2 = 128

NC = 2
NS = 16
NW = NC * NS
ROWS = B * F
R_PER_W = ROWS // NW
CHUNK = 256

VCHUNK = 1024
NVC = 98
V4 = VCHUNK // 4
V4P = NVC * V4
NPK = F * V4P
LCHUNK = 8
NLC = 98
LROWS = F * NLC * LCHUNK


def _repack_body(x_ref, o_ref):
    x = x_ref[0]
    tr = x.T
    t4 = tr.reshape(V4, 4, E)
    o_ref[...] = jnp.concatenate(
        [t4[:, 0, :], t4[:, 1, :], t4[:, 2, :], t4[:, 3, :]], axis=1)


def _repack(fm_t):
    return pl.pallas_call(
        _repack_body,
        grid=(F, NVC),
        in_specs=[pl.BlockSpec((1, E, VCHUNK), lambda f, c: (f, 0, c))],
        out_specs=pl.BlockSpec((V4, 4 * E), lambda f, c: (f * NVC + c, 0)),
        out_shape=jax.ShapeDtypeStruct((NPK, 4 * E), jnp.float32),
    )(fm_t)


def _linpack_body(x_ref, o_ref):
    x = x_ref[0]
    o_ref[...] = jnp.concatenate(
        [x[:, k * 128:(k + 1) * 128] for k in range(LCHUNK)], axis=0)


def _linpack(lin_t3):
    return pl.pallas_call(
        _linpack_body,
        grid=(F, NLC),
        in_specs=[pl.BlockSpec((1, 1, LCHUNK * 128), lambda f, c: (f, 0, c))],
        out_specs=pl.BlockSpec((LCHUNK, 128), lambda f, c: (f * NLC + c, 0)),
        out_shape=jax.ShapeDtypeStruct((LROWS, 128), jnp.float32),
    )(lin_t3)


def _sc_gather_body(r_hbm, le_hbm, s_hbm, pk_hbm, lpk_hbm,
                    g_out, lv_out,
                    r_v, le_v, s_v, rows4_v, rows_v, lin_v,
                    sem, sem_l):
    wid = lax.axis_index("s") * NC + lax.axis_index("c")
    base = wid * R_PER_W
    pltpu.sync_copy(r_hbm.at[pl.ds(base, R_PER_W)], r_v.at[pl.ds(0, R_PER_W)])
    pltpu.sync_copy(le_hbm.at[pl.ds(base, R_PER_W)], le_v.at[pl.ds(0, R_PER_W)])
    pltpu.sync_copy(s_hbm.at[pl.ds(base, R_PER_W)], s_v.at[pl.ds(0, R_PER_W)])
    cp_l = pltpu.async_copy(lpk_hbm.at[le_v.at[pl.ds(0, R_PER_W)]],
                            lin_v, sem_l)

    n_chunks = R_PER_W // CHUNK

    def do_chunk(c, _):
        off = c * CHUNK
        cp = pltpu.async_copy(
            pk_hbm.at[r_v.at[pl.ds(off, CHUNK)]], rows4_v, sem)
        cp.wait()

        def sel(i, _):
            j32 = s_v[pl.ds(off + i, 16)][0]
            rows_v[i, pl.ds(0, 16)] = rows4_v[i, pl.ds(j32, 16)]
            rows_v[i, pl.ds(16, 16)] = rows4_v[i, pl.ds(j32 + 16, 16)]
            return 0
        lax.fori_loop(0, CHUNK, sel, 0, unroll=2)
        pltpu.sync_copy(rows_v, g_out.at[pl.ds(base + off, CHUNK)])
        return 0
    lax.fori_loop(0, n_chunks, do_chunk, 0)

    cp_l.wait()
    pltpu.sync_copy(lin_v, lv_out.at[pl.ds(base, R_PER_W)])


@jax.jit
def _sc_gather(r_flat, le_flat, s_flat, pk, lpk_flat):
    mesh = plsc.VectorSubcoreMesh(
        core_axis_name="c", subcore_axis_name="s",
        num_cores=NC, num_subcores=NS)
    return pl.kernel(
        _sc_gather_body,
        out_type=(
            jax.ShapeDtypeStruct((ROWS, E), jnp.float32),
            jax.ShapeDtypeStruct((ROWS,), jnp.float32),
        ),
        mesh=mesh,
        scratch_types=[
            pltpu.VMEM((R_PER_W + 16,), jnp.int32),
            pltpu.VMEM((R_PER_W + 16,), jnp.int32),
            pltpu.VMEM((R_PER_W + 16,), jnp.int32),
            pltpu.VMEM((CHUNK, 4 * E), jnp.float32),
            pltpu.VMEM((CHUNK, E), jnp.float32),
            pltpu.VMEM((R_PER_W,), jnp.float32),
            pltpu.SemaphoreType.DMA,
            pltpu.SemaphoreType.DMA,
        ],
    )(r_flat, le_flat, s_flat, pk, lpk_flat)


def _tc_body(g_ref, lv_ref, d_ref, w1a_ref, w1b_ref, b1_ref,
             w2_ref, b2_ref, w3_ref, b3_ref, s_ref, out_ref):
    g = g_ref[...]
    sum_v = jnp.dot(g, s_ref[...], preferred_element_type=jnp.float32)
    sq_of_sum = jnp.sum(sum_v * sum_v, axis=1, keepdims=True)
    sum_of_sq = jnp.sum(g * g, axis=1, keepdims=True)
    fm = 0.5 * (sq_of_sum - sum_of_sq)
    lin = jnp.sum(lv_ref[...], axis=1, keepdims=True)
    h = jnp.dot(g, w1a_ref[...], preferred_element_type=jnp.float32)
    h = h + jnp.dot(d_ref[...], w1b_ref[...], preferred_element_type=jnp.float32)
    h = jnp.maximum(h + b1_ref[...], 0.0)
    h = jnp.dot(h, w2_ref[...], preferred_element_type=jnp.float32)
    h = jnp.maximum(h + b2_ref[...], 0.0)
    dp = jnp.sum(h * w3_ref[...], axis=1, keepdims=True)
    out_ref[...] = jax.nn.sigmoid(dp + b3_ref[0, 0] + fm + lin)


def _tc_deepfm(g, lv, dense, w1a, w1b, b1, w2, b2, w3r, b3, smat, bs):
    grid = (B // bs,)
    full = lambda shape: pl.BlockSpec(shape, lambda i: (0, 0))
    return pl.pallas_call(
        _tc_body,
        grid=grid,
        in_specs=[
            pl.BlockSpec((bs, F * E), lambda i: (i, 0)),
            pl.BlockSpec((bs, F), lambda i: (i, 0)),
            pl.BlockSpec((bs, D), lambda i: (i, 0)),
            full((F * E, H1)),
            full((D, H1)),
            full((1, H1)),
            full((H1, H2)),
            full((1, H2)),
            full((1, H2)),
            full((1, 1)),
            full((F * E, E)),
        ],
        out_specs=pl.BlockSpec((bs, 1), lambda i: (i, 0)),
        out_shape=jax.ShapeDtypeStruct((B, 1), jnp.float32),
    )(g, lv, dense, w1a, w1b, b1, w2, b2, w3r, b3, smat)


def kernel(sparse_indices, dense, fm_tables, lin_tables, W1, b1, W2, b2, W3, b3):
    idx = sparse_indices.astype(jnp.int32)
    f_base = (jnp.arange(F, dtype=jnp.int32) * V4P)[None, :]
    r_flat = (f_base + (idx // 4)).reshape(ROWS)
    s_flat = ((idx % 4) * E).reshape(ROWS)
    lf_base = (jnp.arange(F, dtype=jnp.int32) * (NLC * LCHUNK * 128))[None, :]
    le_flat = (lf_base + idx).reshape(ROWS)

    fm_t = fm_tables.transpose(0, 2, 1)
    pk = _repack(fm_t)
    lpk = _linpack(lin_tables.reshape(F, 1, V))

    g_rows, lin_rows = _sc_gather(r_flat, le_flat, s_flat, pk,
                                  lpk.reshape(LROWS * 128))
    g = g_rows.reshape(B, F * E)
    lv = lin_rows.reshape(B, F)

    w1a = W1[:F * E]
    w1b = W1[F * E:]
    smat = jnp.tile(jnp.eye(E, dtype=jnp.float32), (F, 1))
    return _tc_deepfm(
        g, lv, dense,
        w1a, w1b, b1.reshape(1, H1),
        W2, b2.reshape(1, H2),
        W3.reshape(1, H2), b3.reshape(1, 1),
        smat, bs=512)


# speedup vs baseline: 4.1197x; 1.0337x over previous
import functools

import jax
import jax.numpy as jnp
from jax import lax
from jax.experimental import pallas as pl
from jax.experimental.pallas import tpu as pltpu
from jax.experimental.pallas import tpu_sc as plsc

B = 4096
F = 26
V = 100001
E = 32
D = 13
H1 = 128
H2 = 128

NC = 2
NS = 16
NW = NC * NS
ROWS = B * F
R_PER_W = ROWS // NW
CHUNK = 256

VCHUNK = 1024
NVC = 98
V4 = VCHUNK // 4
V4P = NVC * V4
NPK = F * V4P
LCHUNK = 8
NLC = 98
LROWS = F * NLC * LCHUNK


def _repack_body(x_ref, o_ref):
    x = x_ref[0]
    tr = x.T
    t4 = tr.reshape(V4, 4, E)
    o_ref[...] = jnp.concatenate(
        [t4[:, 0, :], t4[:, 1, :], t4[:, 2, :], t4[:, 3, :]], axis=1)


def _repack(fm_t):
    return pl.pallas_call(
        _repack_body,
        grid=(F, NVC),
        in_specs=[pl.BlockSpec((1, E, VCHUNK), lambda f, c: (f, 0, c))],
        out_specs=pl.BlockSpec((V4, 4 * E), lambda f, c: (f * NVC + c, 0)),
        out_shape=jax.ShapeDtypeStruct((NPK, 4 * E), jnp.float32),
    )(fm_t)


def _linpack_body(x_ref, o_ref):
    x = x_ref[0]
    o_ref[...] = jnp.concatenate(
        [x[:, k * 128:(k + 1) * 128] for k in range(LCHUNK)], axis=0)


def _linpack(lin_t3):
    return pl.pallas_call(
        _linpack_body,
        grid=(F, NLC),
        in_specs=[pl.BlockSpec((1, 1, LCHUNK * 128), lambda f, c: (f, 0, c))],
        out_specs=pl.BlockSpec((LCHUNK, 128), lambda f, c: (f * NLC + c, 0)),
        out_shape=jax.ShapeDtypeStruct((LROWS, 128), jnp.float32),
    )(lin_t3)


def _sc_gather_body(r_hbm, le_hbm, s_hbm, pk_hbm, lpk_hbm,
                    g_out, lv_out,
                    r_v, le_v, s_v, rows4_v, rows_v, lin_v,
                    sem, sem_l):
    wid = lax.axis_index("s") * NC + lax.axis_index("c")
    base = wid * R_PER_W
    pltpu.sync_copy(r_hbm.at[pl.ds(base, R_PER_W)], r_v.at[pl.ds(0, R_PER_W)])
    pltpu.sync_copy(le_hbm.at[pl.ds(base, R_PER_W)], le_v.at[pl.ds(0, R_PER_W)])
    pltpu.sync_copy(s_hbm.at[pl.ds(base, R_PER_W)], s_v.at[pl.ds(0, R_PER_W)])
    cp_l = pltpu.async_copy(lpk_hbm.at[le_v.at[pl.ds(0, R_PER_W)]],
                            lin_v, sem_l)

    n_chunks = R_PER_W // CHUNK

    def do_chunk(c, _):
        off = c * CHUNK
        cp = pltpu.async_copy(
            pk_hbm.at[r_v.at[pl.ds(off, CHUNK)]], rows4_v, sem)
        cp.wait()

        def sel(i, _):
            j32 = s_v[pl.ds(off + i, 16)][0]
            rows_v[i, pl.ds(0, 16)] = rows4_v[i, pl.ds(j32, 16)]
            rows_v[i, pl.ds(16, 16)] = rows4_v[i, pl.ds(j32 + 16, 16)]
            return 0
        lax.fori_loop(0, CHUNK, sel, 0, unroll=2)
        pltpu.sync_copy(rows_v, g_out.at[pl.ds(base + off, CHUNK)])
        return 0
    lax.fori_loop(0, n_chunks, do_chunk, 0)

    cp_l.wait()
    pltpu.sync_copy(lin_v, lv_out.at[pl.ds(base, R_PER_W)])


@jax.jit
def _sc_gather(r_flat, le_flat, s_flat, pk, lpk_flat):
    mesh = plsc.VectorSubcoreMesh(
        core_axis_name="c", subcore_axis_name="s",
        num_cores=NC, num_subcores=NS)
    return pl.kernel(
        _sc_gather_body,
        out_type=(
            jax.ShapeDtypeStruct((ROWS, E), jnp.float32),
            jax.ShapeDtypeStruct((ROWS,), jnp.float32),
        ),
        mesh=mesh,
        scratch_types=[
            pltpu.VMEM((R_PER_W + 16,), jnp.int32),
            pltpu.VMEM((R_PER_W + 16,), jnp.int32),
            pltpu.VMEM((R_PER_W + 16,), jnp.int32),
            pltpu.VMEM((CHUNK, 4 * E), jnp.float32),
            pltpu.VMEM((CHUNK, E), jnp.float32),
            pltpu.VMEM((R_PER_W,), jnp.float32),
            pltpu.SemaphoreType.DMA,
            pltpu.SemaphoreType.DMA,
        ],
    )(r_flat, le_flat, s_flat, pk, lpk_flat)


def _tc_body(g_ref, lv_ref, d_ref, w1a_ref, w1b_ref, b1_ref,
             w2_ref, b2_ref, w3_ref, b3_ref, s_ref, out_ref):
    g = g_ref[...]
    sum_v = jnp.dot(g, s_ref[...], preferred_element_type=jnp.float32)
    sq_of_sum = jnp.sum(sum_v * sum_v, axis=1, keepdims=True)
    sum_of_sq = jnp.sum(g * g, axis=1, keepdims=True)
    fm = 0.5 * (sq_of_sum - sum_of_sq)
    lin = jnp.sum(lv_ref[...], axis=1, keepdims=True)
    h = jnp.dot(g, w1a_ref[...], preferred_element_type=jnp.float32)
    h = h + jnp.dot(d_ref[...], w1b_ref[...], preferred_element_type=jnp.float32)
    h = jnp.maximum(h + b1_ref[...], 0.0)
    h = jnp.dot(h, w2_ref[...], preferred_element_type=jnp.float32)
    h = jnp.maximum(h + b2_ref[...], 0.0)
    dp = jnp.sum(h * w3_ref[...], axis=1, keepdims=True)
    out_ref[...] = jax.nn.sigmoid(dp + b3_ref[0, 0] + fm + lin)


def _tc_deepfm(g, lv, dense, w1a, w1b, b1, w2, b2, w3r, b3, smat, bs):
    grid = (B // bs,)
    full = lambda shape: pl.BlockSpec(shape, lambda i: (0, 0))
    return pl.pallas_call(
        _tc_body,
        grid=grid,
        in_specs=[
            pl.BlockSpec((bs, F * E), lambda i: (i, 0)),
            pl.BlockSpec((bs, F), lambda i: (i, 0)),
            pl.BlockSpec((bs, D), lambda i: (i, 0)),
            full((F * E, H1)),
            full((D, H1)),
            full((1, H1)),
            full((H1, H2)),
            full((1, H2)),
            full((1, H2)),
            full((1, 1)),
            full((F * E, E)),
        ],
        out_specs=pl.BlockSpec((bs, 1), lambda i: (i, 0)),
        out_shape=jax.ShapeDtypeStruct((B, 1), jnp.float32),
    )(g, lv, dense, w1a, w1b, b1, w2, b2, w3r, b3, smat)


def kernel(sparse_indices, dense, fm_tables, lin_tables, W1, b1, W2, b2, W3, b3):
    idx = sparse_indices.astype(jnp.int32)
    f_base = (jnp.arange(F, dtype=jnp.int32) * V4P)[None, :]
    r_flat = (f_base + (idx // 4)).reshape(ROWS)
    s_flat = ((idx % 4) * E).reshape(ROWS)
    lf_base = (jnp.arange(F, dtype=jnp.int32) * (NLC * LCHUNK * 128))[None, :]
    le_flat = (lf_base + idx).reshape(ROWS)

    fmp = jnp.pad(fm_tables, ((0, 0), (0, NVC * VCHUNK - V), (0, 0)))
    pk = fmp.reshape(NPK, 4 * E)
    lpk = _linpack(lin_tables.reshape(F, 1, V))

    g_rows, lin_rows = _sc_gather(r_flat, le_flat, s_flat, pk,
                                  lpk.reshape(LROWS * 128))
    g = g_rows.reshape(B, F * E)
    lv = lin_rows.reshape(B, F)

    w1a = W1[:F * E]
    w1b = W1[F * E:]
    smat = jnp.tile(jnp.eye(E, dtype=jnp.float32), (F, 1))
    return _tc_deepfm(
        g, lv, dense,
        w1a, w1b, b1.reshape(1, H1),
        W2, b2.reshape(1, H2),
        W3.reshape(1, H2), b3.reshape(1, 1),
        smat, bs=512)


# speedup vs baseline: 4.9655x; 1.2053x over previous
import functools

import jax
import jax.numpy as jnp
from jax import lax
from jax.experimental import pallas as pl
from jax.experimental.pallas import tpu as pltpu
from jax.experimental.pallas import tpu_sc as plsc

B = 4096
F = 26
V = 100001
E = 32
D = 13
H1 = 128
H2 = 128

NC = 2
NS = 16
NW = NC * NS
ROWS = B * F
R_PER_W = ROWS // NW
CHUNK = 256

VCHUNK = 2048
NVC = 49
V4 = VCHUNK // 4
V4P = NVC * V4
NPK = F * V4P
LCHUNK = 8
NLC = 98
LROWS = F * NLC * LCHUNK


def _repack_body(x_ref, o_ref):
    x = x_ref[0]
    tr = x.T
    t4 = tr.reshape(V4, 4, E)
    o_ref[...] = jnp.concatenate(
        [t4[:, 0, :], t4[:, 1, :], t4[:, 2, :], t4[:, 3, :]], axis=1)


def _repack(fm_t):
    return pl.pallas_call(
        _repack_body,
        grid=(F, NVC),
        in_specs=[pl.BlockSpec((1, E, VCHUNK), lambda f, c: (f, 0, c))],
        out_specs=pl.BlockSpec((V4, 4 * E), lambda f, c: (f * NVC + c, 0)),
        out_shape=jax.ShapeDtypeStruct((NPK, 4 * E), jnp.float32),
    )(fm_t)


def _linpack_body(x_ref, o_ref):
    x = x_ref[0]
    o_ref[...] = jnp.concatenate(
        [x[:, k * 128:(k + 1) * 128] for k in range(LCHUNK)], axis=0)


def _linpack(lin_t3):
    return pl.pallas_call(
        _linpack_body,
        grid=(F, NLC),
        in_specs=[pl.BlockSpec((1, 1, LCHUNK * 128), lambda f, c: (f, 0, c))],
        out_specs=pl.BlockSpec((LCHUNK, 128), lambda f, c: (f * NLC + c, 0)),
        out_shape=jax.ShapeDtypeStruct((LROWS, 128), jnp.float32),
    )(lin_t3)


def _sc_gather_body(r_hbm, le_hbm, s_hbm, pk_hbm, lpk_hbm,
                    g_out, lv_out,
                    r_v, le_v, s_v, rows4_v, rows_v, lin_v,
                    sem, sem_l):
    wid = lax.axis_index("s") * NC + lax.axis_index("c")
    base = wid * R_PER_W
    pltpu.sync_copy(r_hbm.at[pl.ds(base, R_PER_W)], r_v.at[pl.ds(0, R_PER_W)])
    pltpu.sync_copy(le_hbm.at[pl.ds(base, R_PER_W)], le_v.at[pl.ds(0, R_PER_W)])
    pltpu.sync_copy(s_hbm.at[pl.ds(base, R_PER_W)], s_v.at[pl.ds(0, R_PER_W)])
    cp_l = pltpu.async_copy(lpk_hbm.at[le_v.at[pl.ds(0, R_PER_W)]],
                            lin_v, sem_l)

    n_chunks = R_PER_W // CHUNK

    def do_chunk(c, _):
        off = c * CHUNK
        cp = pltpu.async_copy(
            pk_hbm.at[r_v.at[pl.ds(off, CHUNK)]], rows4_v, sem)
        cp.wait()

        def sel(i, _):
            j32 = s_v[pl.ds(off + i, 16)][0]
            rows_v[i, pl.ds(0, 16)] = rows4_v[i, pl.ds(j32, 16)]
            rows_v[i, pl.ds(16, 16)] = rows4_v[i, pl.ds(j32 + 16, 16)]
            return 0
        lax.fori_loop(0, CHUNK, sel, 0, unroll=2)
        pltpu.sync_copy(rows_v, g_out.at[pl.ds(base + off, CHUNK)])
        return 0
    lax.fori_loop(0, n_chunks, do_chunk, 0)

    cp_l.wait()
    pltpu.sync_copy(lin_v, lv_out.at[pl.ds(base, R_PER_W)])


@jax.jit
def _sc_gather(r_flat, le_flat, s_flat, pk, lpk_flat):
    mesh = plsc.VectorSubcoreMesh(
        core_axis_name="c", subcore_axis_name="s",
        num_cores=NC, num_subcores=NS)
    return pl.kernel(
        _sc_gather_body,
        out_type=(
            jax.ShapeDtypeStruct((ROWS, E), jnp.float32),
            jax.ShapeDtypeStruct((ROWS,), jnp.float32),
        ),
        mesh=mesh,
        scratch_types=[
            pltpu.VMEM((R_PER_W + 16,), jnp.int32),
            pltpu.VMEM((R_PER_W + 16,), jnp.int32),
            pltpu.VMEM((R_PER_W + 16,), jnp.int32),
            pltpu.VMEM((CHUNK, 4 * E), jnp.float32),
            pltpu.VMEM((CHUNK, E), jnp.float32),
            pltpu.VMEM((R_PER_W,), jnp.float32),
            pltpu.SemaphoreType.DMA,
            pltpu.SemaphoreType.DMA,
        ],
    )(r_flat, le_flat, s_flat, pk, lpk_flat)


def _tc_body(g_ref, lv_ref, d_ref, w1a_ref, w1b_ref, b1_ref,
             w2_ref, b2_ref, w3_ref, b3_ref, s_ref, out_ref):
    g = g_ref[...]
    sum_v = jnp.dot(g, s_ref[...], preferred_element_type=jnp.float32)
    sq_of_sum = jnp.sum(sum_v * sum_v, axis=1, keepdims=True)
    sum_of_sq = jnp.sum(g * g, axis=1, keepdims=True)
    fm = 0.5 * (sq_of_sum - sum_of_sq)
    lin = jnp.sum(lv_ref[...], axis=1, keepdims=True)
    h = jnp.dot(g, w1a_ref[...], preferred_element_type=jnp.float32)
    h = h + jnp.dot(d_ref[...], w1b_ref[...], preferred_element_type=jnp.float32)
    h = jnp.maximum(h + b1_ref[...], 0.0)
    h = jnp.dot(h, w2_ref[...], preferred_element_type=jnp.float32)
    h = jnp.maximum(h + b2_ref[...], 0.0)
    dp = jnp.sum(h * w3_ref[...], axis=1, keepdims=True)
    out_ref[...] = jax.nn.sigmoid(dp + b3_ref[0, 0] + fm + lin)


def _tc_deepfm(g, lv, dense, w1a, w1b, b1, w2, b2, w3r, b3, smat, bs):
    grid = (B // bs,)
    full = lambda shape: pl.BlockSpec(shape, lambda i: (0, 0))
    return pl.pallas_call(
        _tc_body,
        grid=grid,
        in_specs=[
            pl.BlockSpec((bs, F * E), lambda i: (i, 0)),
            pl.BlockSpec((bs, F), lambda i: (i, 0)),
            pl.BlockSpec((bs, D), lambda i: (i, 0)),
            full((F * E, H1)),
            full((D, H1)),
            full((1, H1)),
            full((H1, H2)),
            full((1, H2)),
            full((1, H2)),
            full((1, 1)),
            full((F * E, E)),
        ],
        out_specs=pl.BlockSpec((bs, 1), lambda i: (i, 0)),
        out_shape=jax.ShapeDtypeStruct((B, 1), jnp.float32),
    )(g, lv, dense, w1a, w1b, b1, w2, b2, w3r, b3, smat)


def kernel(sparse_indices, dense, fm_tables, lin_tables, W1, b1, W2, b2, W3, b3):
    idx = sparse_indices.astype(jnp.int32)
    f_base = (jnp.arange(F, dtype=jnp.int32) * V4P)[None, :]
    r_flat = (f_base + (idx // 4)).reshape(ROWS)
    s_flat = ((idx % 4) * E).reshape(ROWS)
    lf_base = (jnp.arange(F, dtype=jnp.int32) * (NLC * LCHUNK * 128))[None, :]
    le_flat = (lf_base + idx).reshape(ROWS)

    fm_t = fm_tables.transpose(0, 2, 1)
    pk = _repack(fm_t)
    lpk = _linpack(lin_tables.reshape(F, 1, V))

    g_rows, lin_rows = _sc_gather(r_flat, le_flat, s_flat, pk,
                                  lpk.reshape(LROWS * 128))
    g = g_rows.reshape(B, F * E)
    lv = lin_rows.reshape(B, F)

    w1a = W1[:F * E]
    w1b = W1[F * E:]
    smat = jnp.tile(jnp.eye(E, dtype=jnp.float32), (F, 1))
    return _tc_deepfm(
        g, lv, dense,
        w1a, w1b, b1.reshape(1, H1),
        W2, b2.reshape(1, H2),
        W3.reshape(1, H2), b3.reshape(1, 1),
        smat, bs=512)


# speedup vs baseline: 5.3296x; 1.0733x over previous
import functools

import jax
import jax.numpy as jnp
from jax import lax
from jax.experimental import pallas as pl
from jax.experimental.pallas import tpu as pltpu
from jax.experimental.pallas import tpu_sc as plsc

B = 4096
F = 26
V = 100001
E = 32
D = 13
H1 = 128
H2 = 128

NC = 2
NS = 16
NW = NC * NS
ROWS = B * F
R_PER_W = ROWS // NW
CHUNK = 256

VCHUNK = 4096
NVC = 25
V4 = VCHUNK // 4
V4P = NVC * V4
NPK = F * V4P
LCHUNK = 8
NLC = 98
LROWS = F * NLC * LCHUNK


def _repack_body(x_ref, o_ref):
    x = x_ref[0]
    tr = x.T
    t4 = tr.reshape(V4, 4, E)
    o_ref[...] = jnp.concatenate(
        [t4[:, 0, :], t4[:, 1, :], t4[:, 2, :], t4[:, 3, :]], axis=1)


def _repack(fm_t):
    return pl.pallas_call(
        _repack_body,
        grid=(F, NVC),
        in_specs=[pl.BlockSpec((1, E, VCHUNK), lambda f, c: (f, 0, c))],
        out_specs=pl.BlockSpec((V4, 4 * E), lambda f, c: (f * NVC + c, 0)),
        out_shape=jax.ShapeDtypeStruct((NPK, 4 * E), jnp.float32),
    )(fm_t)


def _linpack_body(x_ref, o_ref):
    x = x_ref[0]
    o_ref[...] = jnp.concatenate(
        [x[:, k * 128:(k + 1) * 128] for k in range(LCHUNK)], axis=0)


def _linpack(lin_t3):
    return pl.pallas_call(
        _linpack_body,
        grid=(F, NLC),
        in_specs=[pl.BlockSpec((1, 1, LCHUNK * 128), lambda f, c: (f, 0, c))],
        out_specs=pl.BlockSpec((LCHUNK, 128), lambda f, c: (f * NLC + c, 0)),
        out_shape=jax.ShapeDtypeStruct((LROWS, 128), jnp.float32),
    )(lin_t3)


def _sc_gather_body(r_hbm, le_hbm, s_hbm, pk_hbm, lpk_hbm,
                    g_out, lv_out,
                    r_v, le_v, s_v, rows4_v, rows_v, lin_v,
                    sem, sem_l):
    wid = lax.axis_index("s") * NC + lax.axis_index("c")
    base = wid * R_PER_W
    pltpu.sync_copy(r_hbm.at[pl.ds(base, R_PER_W)], r_v.at[pl.ds(0, R_PER_W)])
    pltpu.sync_copy(le_hbm.at[pl.ds(base, R_PER_W)], le_v.at[pl.ds(0, R_PER_W)])
    pltpu.sync_copy(s_hbm.at[pl.ds(base, R_PER_W)], s_v.at[pl.ds(0, R_PER_W)])
    cp_l = pltpu.async_copy(lpk_hbm.at[le_v.at[pl.ds(0, R_PER_W)]],
                            lin_v, sem_l)

    n_chunks = R_PER_W // CHUNK

    def do_chunk(c, _):
        off = c * CHUNK
        cp = pltpu.async_copy(
            pk_hbm.at[r_v.at[pl.ds(off, CHUNK)]], rows4_v, sem)
        cp.wait()

        def sel(i, _):
            j32 = s_v[pl.ds(off + i, 16)][0]
            rows_v[i, pl.ds(0, 16)] = rows4_v[i, pl.ds(j32, 16)]
            rows_v[i, pl.ds(16, 16)] = rows4_v[i, pl.ds(j32 + 16, 16)]
            return 0
        lax.fori_loop(0, CHUNK, sel, 0, unroll=2)
        pltpu.sync_copy(rows_v, g_out.at[pl.ds(base + off, CHUNK)])
        return 0
    lax.fori_loop(0, n_chunks, do_chunk, 0)

    cp_l.wait()
    pltpu.sync_copy(lin_v, lv_out.at[pl.ds(base, R_PER_W)])


@jax.jit
def _sc_gather(r_flat, le_flat, s_flat, pk, lpk_flat):
    mesh = plsc.VectorSubcoreMesh(
        core_axis_name="c", subcore_axis_name="s",
        num_cores=NC, num_subcores=NS)
    return pl.kernel(
        _sc_gather_body,
        out_type=(
            jax.ShapeDtypeStruct((ROWS, E), jnp.float32),
            jax.ShapeDtypeStruct((ROWS,), jnp.float32),
        ),
        mesh=mesh,
        scratch_types=[
            pltpu.VMEM((R_PER_W + 16,), jnp.int32),
            pltpu.VMEM((R_PER_W + 16,), jnp.int32),
            pltpu.VMEM((R_PER_W + 16,), jnp.int32),
            pltpu.VMEM((CHUNK, 4 * E), jnp.float32),
            pltpu.VMEM((CHUNK, E), jnp.float32),
            pltpu.VMEM((R_PER_W,), jnp.float32),
            pltpu.SemaphoreType.DMA,
            pltpu.SemaphoreType.DMA,
        ],
    )(r_flat, le_flat, s_flat, pk, lpk_flat)


def _tc_body(g_ref, lv_ref, d_ref, w1a_ref, w1b_ref, b1_ref,
             w2_ref, b2_ref, w3_ref, b3_ref, s_ref, out_ref):
    g = g_ref[...]
    sum_v = jnp.dot(g, s_ref[...], preferred_element_type=jnp.float32)
    sq_of_sum = jnp.sum(sum_v * sum_v, axis=1, keepdims=True)
    sum_of_sq = jnp.sum(g * g, axis=1, keepdims=True)
    fm = 0.5 * (sq_of_sum - sum_of_sq)
    lin = jnp.sum(lv_ref[...], axis=1, keepdims=True)
    h = jnp.dot(g, w1a_ref[...], preferred_element_type=jnp.float32)
    h = h + jnp.dot(d_ref[...], w1b_ref[...], preferred_element_type=jnp.float32)
    h = jnp.maximum(h + b1_ref[...], 0.0)
    h = jnp.dot(h, w2_ref[...], preferred_element_type=jnp.float32)
    h = jnp.maximum(h + b2_ref[...], 0.0)
    dp = jnp.sum(h * w3_ref[...], axis=1, keepdims=True)
    out_ref[...] = jax.nn.sigmoid(dp + b3_ref[0, 0] + fm + lin)


def _tc_deepfm(g, lv, dense, w1a, w1b, b1, w2, b2, w3r, b3, smat, bs):
    grid = (B // bs,)
    full = lambda shape: pl.BlockSpec(shape, lambda i: (0, 0))
    return pl.pallas_call(
        _tc_body,
        grid=grid,
        in_specs=[
            pl.BlockSpec((bs, F * E), lambda i: (i, 0)),
            pl.BlockSpec((bs, F), lambda i: (i, 0)),
            pl.BlockSpec((bs, D), lambda i: (i, 0)),
            full((F * E, H1)),
            full((D, H1)),
            full((1, H1)),
            full((H1, H2)),
            full((1, H2)),
            full((1, H2)),
            full((1, 1)),
            full((F * E, E)),
        ],
        out_specs=pl.BlockSpec((bs, 1), lambda i: (i, 0)),
        out_shape=jax.ShapeDtypeStruct((B, 1), jnp.float32),
    )(g, lv, dense, w1a, w1b, b1, w2, b2, w3r, b3, smat)


def kernel(sparse_indices, dense, fm_tables, lin_tables, W1, b1, W2, b2, W3, b3):
    idx = sparse_indices.astype(jnp.int32)
    f_base = (jnp.arange(F, dtype=jnp.int32) * V4P)[None, :]
    r_flat = (f_base + (idx // 4)).reshape(ROWS)
    s_flat = ((idx % 4) * E).reshape(ROWS)
    lf_base = (jnp.arange(F, dtype=jnp.int32) * (NLC * LCHUNK * 128))[None, :]
    le_flat = (lf_base + idx).reshape(ROWS)

    fm_t = fm_tables.transpose(0, 2, 1)
    pk = _repack(fm_t)
    lpk = _linpack(lin_tables.reshape(F, 1, V))

    g_rows, lin_rows = _sc_gather(r_flat, le_flat, s_flat, pk,
                                  lpk.reshape(LROWS * 128))
    g = g_rows.reshape(B, F * E)
    lv = lin_rows.reshape(B, F)

    w1a = W1[:F * E]
    w1b = W1[F * E:]
    smat = jnp.tile(jnp.eye(E, dtype=jnp.float32), (F, 1))
    return _tc_deepfm(
        g, lv, dense,
        w1a, w1b, b1.reshape(1, H1),
        W2, b2.reshape(1, H2),
        W3.reshape(1, H2), b3.reshape(1, 1),
        smat, bs=512)
